# baseline (speedup 1.0000x reference)
import functools

import jax
import jax.numpy as jnp
from jax import lax
from jax.experimental import pallas as pl
from jax.experimental.pallas import tpu as pltpu
from jax.experimental.pallas import tpu_sc as plsc

_EMB = 32
_B = 16384
_NUMF = 16
_HID = 128
_NCLS = 2
_NC, _NS = 2, 16
_NW = _NC * _NS
_BPW = _B // _NW
_CH = 128
_NCH = _BPW // _CH

def _gather_body(src_id, tgt_id, src_emb, tgt_emb, src_out, tgt_out,
                 sidx, tidx, srows, trows, sem):
    wid = lax.axis_index("s") * _NC + lax.axis_index("c")
    pltpu.sync_copy(src_id.at[wid], sidx)
    pltpu.sync_copy(tgt_id.at[wid], tidx)
    copies = []
    for j in range(_NCH):
        copies.append(pltpu.async_copy(src_emb.at[sidx.at[j]], srows.at[j], sem))
        copies.append(pltpu.async_copy(tgt_emb.at[tidx.at[j]], trows.at[j], sem))
    for c in copies:
        c.wait()
    pltpu.sync_copy(srows, src_out.at[wid])
    pltpu.sync_copy(trows, tgt_out.at[wid])


_gather_cache = []


def _gather(*args):
    if not _gather_cache:
        mesh = plsc.VectorSubcoreMesh(
            core_axis_name="c", subcore_axis_name="s",
            num_cores=_NC, num_subcores=_NS,
        )
        _gather_cache.append(pl.kernel(
            _gather_body,
            out_type=(
                jax.ShapeDtypeStruct((_NW, _NCH, _CH, _EMB), jnp.float32),
                jax.ShapeDtypeStruct((_NW, _NCH, _CH, _EMB), jnp.float32),
            ),
            mesh=mesh,
            scratch_types=[
                pltpu.VMEM((_NCH, _CH), jnp.int32),
                pltpu.VMEM((_NCH, _CH), jnp.int32),
                pltpu.VMEM((_NCH, _CH, _EMB), jnp.float32),
                pltpu.VMEM((_NCH, _CH, _EMB), jnp.float32),
                pltpu.SemaphoreType.DMA,
            ],
            compiler_params=pltpu.CompilerParams(use_tc_tiling_on_sc=False),
        ))
    return _gather_cache[0](*args)


def _mlp_body(s, t, n, w1s, w1t, w1n, b1, w2, b2, o):
    h = (jnp.dot(s[...], w1s[...], preferred_element_type=jnp.float32)
         + jnp.dot(t[...], w1t[...], preferred_element_type=jnp.float32)
         + jnp.dot(n[...], w1n[...], preferred_element_type=jnp.float32)
         + b1[...])
    h = jnp.maximum(h, 0.0)
    o[...] = jnp.dot(h, w2[...], preferred_element_type=jnp.float32) + b2[...]


_BLK = 2048


def _mlp(s, t, n, w1s, w1t, w1n, b1, w2, b2):
    grid = (_B // _BLK,)
    full = lambda i: (0, 0)
    return pl.pallas_call(
        _mlp_body,
        grid=grid,
        in_specs=[
            pl.BlockSpec((_BLK, _EMB), lambda i: (i, 0)),
            pl.BlockSpec((_BLK, _EMB), lambda i: (i, 0)),
            pl.BlockSpec((_BLK, _NUMF), lambda i: (i, 0)),
            pl.BlockSpec((_EMB, _HID), full),
            pl.BlockSpec((_EMB, _HID), full),
            pl.BlockSpec((_NUMF, _HID), full),
            pl.BlockSpec((1, _HID), full),
            pl.BlockSpec((_HID, _NCLS), full),
            pl.BlockSpec((1, _NCLS), full),
        ],
        out_specs=pl.BlockSpec((_BLK, _NCLS), lambda i: (i, 0)),
        out_shape=jax.ShapeDtypeStruct((_B, _NCLS), jnp.float32),
    )(s, t, n, w1s, w1t, w1n, b1, w2, b2)


def kernel(cat_feats, num_feats, src_emb, tgt_emb, W1, b1, W2, b2):
    src_id = cat_feats[:, 0].reshape(_NW, _NCH, _CH)
    tgt_id = cat_feats[:, 1].reshape(_NW, _NCH, _CH)
    srows, trows = _gather(src_id, tgt_id, src_emb, tgt_emb)
    s = srows.reshape(_B, _EMB)
    t = trows.reshape(_B, _EMB)
    w1s = W1[:, :_EMB].T
    w1t = W1[:, _EMB:2 * _EMB].T
    w1n = W1[:, 2 * _EMB:].T
    return _mlp(s, t, num_feats, w1s, w1t, w1n,
                b1.reshape(1, _HID), W2.T, b2.reshape(1, _NCLS))
